# Initial kernel scaffold; baseline (speedup 1.0000x reference)
import jax
import jax.numpy as jnp
from jax.experimental import pallas as pl


def kernel(x, edge_index, enc_W1, enc_b1, enc_W2, enc_b2, conv1_W, conv1_att_src, conv1_att_dst, conv1_bias, conv2_W, conv2_att_src, conv2_att_dst, conv2_bias, conv3_W, conv3_att_src, conv3_att_dst, conv3_bias, out_W1, out_b1, out_W2, out_b2, out_W3, out_b3):
    raise NotImplementedError("write your pallas kernel here")


# speedup vs baseline: 1.0486x; 1.0486x over previous
import functools

import jax
import jax.numpy as jnp
from jax.experimental import pallas as pl
from jax.experimental.pallas import tpu as pltpu

N = 50000
E = 800000
ETOT = E + N
H = 4
D = 64
HD = H * D

_BN = 2000


def _elu(v):
    return jnp.where(v > 0, v, jnp.exp(v) - 1.0)


def _enc_body(x_ref, w1_ref, b1_ref, w2_ref, b2_ref, o_ref):
    h = jnp.dot(x_ref[...], w1_ref[...], preferred_element_type=jnp.float32)
    h = _elu(h + b1_ref[...][None, :])
    h = jnp.dot(h, w2_ref[...], preferred_element_type=jnp.float32)
    o_ref[...] = _elu(h + b2_ref[...][None, :])


def _encoder(x, w1, b1, w2, b2):
    return pl.pallas_call(
        _enc_body,
        grid=(N // _BN,),
        in_specs=[
            pl.BlockSpec((_BN, 8), lambda i: (i, 0)),
            pl.BlockSpec((8, 32), lambda i: (0, 0)),
            pl.BlockSpec((32,), lambda i: (0,)),
            pl.BlockSpec((32, D), lambda i: (0, 0)),
            pl.BlockSpec((D,), lambda i: (0,)),
        ],
        out_specs=pl.BlockSpec((_BN, D), lambda i: (i, 0)),
        out_shape=jax.ShapeDtypeStruct((N, D), jnp.float32),
    )(x, w1, b1, w2, b2)


def _proj_body(h_ref, w_ref, asw_ref, adw_ref, hf_ref, as_ref, ad_ref, m_ref):
    i = pl.program_id(0)
    hf = jnp.dot(h_ref[...], w_ref[...], preferred_element_type=jnp.float32)
    hf_ref[...] = hf
    hh = hf.reshape(_BN, H, D)
    a_s = jnp.sum(hh * asw_ref[...][None, :, :], axis=-1)
    a_d = jnp.sum(hh * adw_ref[...][None, :, :], axis=-1)
    as_ref[...] = a_s
    ad_ref[...] = a_d
    blk = jnp.concatenate([jnp.max(a_s, axis=0)[None, :],
                           jnp.max(a_d, axis=0)[None, :]], axis=0)

    @pl.when(i == 0)
    def _():
        m_ref[...] = blk

    @pl.when(i > 0)
    def _():
        m_ref[...] = jnp.maximum(m_ref[...], blk)


def _projection(h, w, att_src, att_dst):
    return pl.pallas_call(
        _proj_body,
        grid=(N // _BN,),
        in_specs=[
            pl.BlockSpec((_BN, D), lambda i: (i, 0)),
            pl.BlockSpec((D, HD), lambda i: (0, 0)),
            pl.BlockSpec((H, D), lambda i: (0, 0)),
            pl.BlockSpec((H, D), lambda i: (0, 0)),
        ],
        out_specs=[
            pl.BlockSpec((_BN, HD), lambda i: (i, 0)),
            pl.BlockSpec((_BN, H), lambda i: (i, 0)),
            pl.BlockSpec((_BN, H), lambda i: (i, 0)),
            pl.BlockSpec((2, H), lambda i: (0, 0)),
        ],
        out_shape=[
            jax.ShapeDtypeStruct((N, HD), jnp.float32),
            jax.ShapeDtypeStruct((N, H), jnp.float32),
            jax.ShapeDtypeStruct((N, H), jnp.float32),
            jax.ShapeDtypeStruct((2, H), jnp.float32),
        ],
    )(h, w, att_src.reshape(H, D), att_dst.reshape(H, D))


def _dec_body(h_ref, w1_ref, b1_ref, w2_ref, b2_ref, w3_ref, b3_ref, o_ref):
    o = _elu(jnp.dot(h_ref[...], w1_ref[...],
                     preferred_element_type=jnp.float32) + b1_ref[...][None, :])
    o = _elu(jnp.dot(o, w2_ref[...],
                     preferred_element_type=jnp.float32) + b2_ref[...][None, :])
    o_ref[...] = jnp.dot(o, w3_ref[...],
                         preferred_element_type=jnp.float32) + b3_ref[...][None, :]


def _decoder(h, w1, b1, w2, b2, w3, b3):
    return pl.pallas_call(
        _dec_body,
        grid=(N // _BN,),
        in_specs=[
            pl.BlockSpec((_BN, D), lambda i: (i, 0)),
            pl.BlockSpec((D, 64), lambda i: (0, 0)),
            pl.BlockSpec((64,), lambda i: (0,)),
            pl.BlockSpec((64, 32), lambda i: (0, 0)),
            pl.BlockSpec((32,), lambda i: (0,)),
            pl.BlockSpec((32, 8), lambda i: (0, 0)),
            pl.BlockSpec((8,), lambda i: (0,)),
        ],
        out_specs=pl.BlockSpec((_BN, 8), lambda i: (i, 0)),
        out_shape=jax.ShapeDtypeStruct((N, 8), jnp.float32),
    )(h, w1, b1, w2, b2, w3, b3)


def _edge_phase(hf, a_src, a_dst, m_bound, src, dst, bias, h_prev):
    mh = jax.nn.leaky_relu(m_bound[0] + m_bound[1], 0.2)
    e = jax.nn.leaky_relu(a_src[src] + a_dst[dst], 0.2)
    ex = jnp.exp(e - mh[None, :])
    den = jax.ops.segment_sum(ex, dst, num_segments=N)
    alpha = ex / (den[dst] + 1e-16)
    hh = hf.reshape(N, H, D)
    msg = hh[src] * alpha[:, :, None]
    agg = jax.ops.segment_sum(msg, dst, num_segments=N)
    return jnp.mean(agg, axis=1) + bias + h_prev


def kernel(x, edge_index, enc_W1, enc_b1, enc_W2, enc_b2,
           conv1_W, conv1_att_src, conv1_att_dst, conv1_bias,
           conv2_W, conv2_att_src, conv2_att_dst, conv2_bias,
           conv3_W, conv3_att_src, conv3_att_dst, conv3_bias,
           out_W1, out_b1, out_W2, out_b2, out_W3, out_b3):
    loop = jnp.arange(N, dtype=edge_index.dtype)
    src = jnp.concatenate([edge_index[0], loop])
    dst = jnp.concatenate([edge_index[1], loop])

    h = _encoder(x, enc_W1, enc_b1, enc_W2, enc_b2)
    for (w, asw, adw, b) in (
            (conv1_W, conv1_att_src, conv1_att_dst, conv1_bias),
            (conv2_W, conv2_att_src, conv2_att_dst, conv2_bias),
            (conv3_W, conv3_att_src, conv3_att_dst, conv3_bias)):
        hf, a_s, a_d, m = _projection(h, w, asw, adw)
        h = _edge_phase(hf, a_s, a_d, m, src, dst, b, h)
    return _decoder(h, out_W1, out_b1, out_W2, out_b2, out_W3, out_b3)


# speedup vs baseline: 1.0871x; 1.0368x over previous
import functools

import jax
import jax.numpy as jnp
from jax import lax
from jax.experimental import pallas as pl
from jax.experimental.pallas import tpu as pltpu
from jax.experimental.pallas import tpu_sc as plsc

N = 50000
E = 800000
ETOT = E + N
H = 4
D = 64
HD = H * D

NP = 53248
TA = 26624
EP = 32 * TA
_NBATCH = TA // 1024

_BN = 2000


def _elu(v):
    return jnp.where(v > 0, v, jnp.exp(v) - 1.0)


def _enc_body(x_ref, w1_ref, b1_ref, w2_ref, b2_ref, o_ref):
    h = jnp.dot(x_ref[...], w1_ref[...], preferred_element_type=jnp.float32)
    h = _elu(h + b1_ref[...][None, :])
    h = jnp.dot(h, w2_ref[...], preferred_element_type=jnp.float32)
    o_ref[...] = _elu(h + b2_ref[...][None, :])


def _encoder(x, w1, b1, w2, b2):
    return pl.pallas_call(
        _enc_body,
        grid=(N // _BN,),
        in_specs=[
            pl.BlockSpec((_BN, 8), lambda i: (i, 0)),
            pl.BlockSpec((8, 32), lambda i: (0, 0)),
            pl.BlockSpec((32,), lambda i: (0,)),
            pl.BlockSpec((32, D), lambda i: (0, 0)),
            pl.BlockSpec((D,), lambda i: (0,)),
        ],
        out_specs=pl.BlockSpec((_BN, D), lambda i: (i, 0)),
        out_shape=jax.ShapeDtypeStruct((N, D), jnp.float32),
    )(x, w1, b1, w2, b2)


def _proj_body(h_ref, w_ref, asw_ref, adw_ref, hf_ref, as_ref, ad_ref, m_ref):
    i = pl.program_id(0)
    hf = jnp.dot(h_ref[...], w_ref[...], preferred_element_type=jnp.float32)
    hf_ref[...] = hf
    hh = hf.reshape(_BN, H, D)
    a_s = jnp.sum(hh * asw_ref[...][None, :, :], axis=-1)
    a_d = jnp.sum(hh * adw_ref[...][None, :, :], axis=-1)
    as_ref[...] = a_s
    ad_ref[...] = a_d
    blk = jnp.concatenate([jnp.max(a_s, axis=0), jnp.max(a_d, axis=0),
                           jnp.zeros((24,), jnp.float32)])

    @pl.when(i == 0)
    def _():
        m_ref[...] = blk

    @pl.when(i > 0)
    def _():
        m_ref[...] = jnp.maximum(m_ref[...], blk)


def _projection(h, w, att_src, att_dst):
    return pl.pallas_call(
        _proj_body,
        grid=(N // _BN,),
        in_specs=[
            pl.BlockSpec((_BN, D), lambda i: (i, 0)),
            pl.BlockSpec((D, HD), lambda i: (0, 0)),
            pl.BlockSpec((H, D), lambda i: (0, 0)),
            pl.BlockSpec((H, D), lambda i: (0, 0)),
        ],
        out_specs=[
            pl.BlockSpec((_BN, HD), lambda i: (i, 0)),
            pl.BlockSpec((_BN, H), lambda i: (i, 0)),
            pl.BlockSpec((_BN, H), lambda i: (i, 0)),
            pl.BlockSpec((32,), lambda i: (0,)),
        ],
        out_shape=[
            jax.ShapeDtypeStruct((N, HD), jnp.float32),
            jax.ShapeDtypeStruct((N, H), jnp.float32),
            jax.ShapeDtypeStruct((N, H), jnp.float32),
            jax.ShapeDtypeStruct((32,), jnp.float32),
        ],
    )(h, w, att_src.reshape(H, D), att_dst.reshape(H, D))


def _dec_body(h_ref, w1_ref, b1_ref, w2_ref, b2_ref, w3_ref, b3_ref, o_ref):
    o = _elu(jnp.dot(h_ref[...], w1_ref[...],
                     preferred_element_type=jnp.float32) + b1_ref[...][None, :])
    o = _elu(jnp.dot(o, w2_ref[...],
                     preferred_element_type=jnp.float32) + b2_ref[...][None, :])
    o_ref[...] = jnp.dot(o, w3_ref[...],
                         preferred_element_type=jnp.float32) + b3_ref[...][None, :]


def _decoder(h, w1, b1, w2, b2, w3, b3):
    return pl.pallas_call(
        _dec_body,
        grid=(N // _BN,),
        in_specs=[
            pl.BlockSpec((_BN, D), lambda i: (i, 0)),
            pl.BlockSpec((D, 64), lambda i: (0, 0)),
            pl.BlockSpec((64,), lambda i: (0,)),
            pl.BlockSpec((64, 32), lambda i: (0, 0)),
            pl.BlockSpec((32,), lambda i: (0,)),
            pl.BlockSpec((32, 8), lambda i: (0, 0)),
            pl.BlockSpec((8,), lambda i: (0,)),
        ],
        out_specs=pl.BlockSpec((_BN, 8), lambda i: (i, 0)),
        out_shape=jax.ShapeDtypeStruct((N, 8), jnp.float32),
    )(h, w1, b1, w2, b2, w3, b3)


_SC_MESH = plsc.VectorSubcoreMesh(core_axis_name="c", subcore_axis_name="s")


@functools.partial(
    pl.kernel,
    out_type=[
        jax.ShapeDtypeStruct((H, EP), jnp.float32),
        jax.ShapeDtypeStruct((2 * H * NP,), jnp.float32),
    ],
    mesh=_SC_MESH,
    compiler_params=pltpu.CompilerParams(use_tc_tiling_on_sc=False),
    scratch_types=[
        pltpu.VMEM((1024,), jnp.int32),
        pltpu.VMEM((1024,), jnp.int32),
        pltpu.VMEM((H, 1024), jnp.float32),
        pltpu.VMEM((H, 1024), jnp.float32),
        pltpu.VMEM((H, 1024), jnp.float32),
        pltpu.VMEM((32,), jnp.float32),
        pltpu.VMEM_SHARED((H, NP), jnp.float32),
        pltpu.SemaphoreType.DMA,
        pltpu.SemaphoreType.DMA,
    ],
)
def _attn_sc(src_ref, dst_ref, asrc_ref, adst_ref, m_ref, zeros_ref,
             ex_ref, den_ref,
             srcb, dstb, arow, brow, exb, mb, den_acc, sem1, sem2):
    c = lax.axis_index("c")
    s = lax.axis_index("s")
    wid = s * 2 + c
    rows = NP // 16
    for h in range(H):
        pltpu.sync_copy(zeros_ref.at[pl.ds(s * rows, rows)],
                        den_acc.at[h].at[pl.ds(s * rows, rows)])
    pltpu.sync_copy(m_ref, mb)
    plsc.subcore_barrier()

    mv = mb[pl.ds(0, 16)]
    mh = []
    for h in range(H):
        v = mv[h] + mv[H + h]
        mh.append(jnp.where(v > 0, v, 0.2 * v))
    base0 = wid * TA

    def batch(b, carry):
        base = base0 + b * 1024
        pltpu.sync_copy(src_ref.at[pl.ds(base, 1024)], srcb)
        pltpu.sync_copy(dst_ref.at[pl.ds(base, 1024)], dstb)
        cps = []
        for h in range(H):
            cps.append(pltpu.async_copy(asrc_ref.at[h].at[srcb],
                                        arow.at[h], sem1))
            cps.append(pltpu.async_copy(adst_ref.at[h].at[dstb],
                                        brow.at[h], sem2))
        for cp in cps:
            cp.wait()

        def vec(j, carry2):
            sl = pl.ds(j * 16, 16)
            for h in range(H):
                e = arow[h, sl] + brow[h, sl]
                e = jnp.where(e > 0, e, 0.2 * e) - mh[h]
                exb[h, sl] = jnp.exp(e)
            return carry2

        lax.fori_loop(0, 64, vec, 0)
        for h in range(H):
            pltpu.sync_copy(exb.at[h], ex_ref.at[h].at[pl.ds(base, 1024)])
            pltpu.sync_copy(exb.at[h], den_acc.at[h].at[dstb], add=True)
        return carry

    lax.fori_loop(0, _NBATCH, batch, 0)
    plsc.subcore_barrier()
    for h in range(H):
        pltpu.sync_copy(den_acc.at[h].at[pl.ds(s * rows, rows)],
                        den_ref.at[pl.ds((c * H + h) * NP + s * rows, rows)])


def _edge_phase(hf, a_src, a_dst, m_bound, srcp, dstp, zeros_np, bias, h_prev):
    asrc_t = a_src.T
    adst_t = jnp.pad(a_dst.T, ((0, 0), (0, NP - N)))
    ex, den = _attn_sc(srcp, dstp, asrc_t, adst_t, m_bound, zeros_np)
    den_tot = den.reshape(2, H, NP).sum(axis=0)
    src = srcp[:ETOT]
    dst = dstp[:ETOT]
    alpha = ex[:, :ETOT].T / (den_tot[:, dst].T + 1e-16)
    hh = hf.reshape(N, H, D)
    msg = hh[src] * alpha[:, :, None]
    agg = jax.ops.segment_sum(msg, dst, num_segments=N)
    return jnp.mean(agg, axis=1) + bias + h_prev


def kernel(x, edge_index, enc_W1, enc_b1, enc_W2, enc_b2,
           conv1_W, conv1_att_src, conv1_att_dst, conv1_bias,
           conv2_W, conv2_att_src, conv2_att_dst, conv2_bias,
           conv3_W, conv3_att_src, conv3_att_dst, conv3_bias,
           out_W1, out_b1, out_W2, out_b2, out_W3, out_b3):
    loop = jnp.arange(N, dtype=edge_index.dtype)
    srcp = jnp.concatenate([edge_index[0], loop,
                            jnp.zeros((EP - ETOT,), edge_index.dtype)])
    dstp = jnp.concatenate([edge_index[1], loop,
                            jnp.full((EP - ETOT,), N, edge_index.dtype)])
    zeros_np = jnp.zeros((NP,), jnp.float32)

    h = _encoder(x, enc_W1, enc_b1, enc_W2, enc_b2)
    for (w, asw, adw, b) in (
            (conv1_W, conv1_att_src, conv1_att_dst, conv1_bias),
            (conv2_W, conv2_att_src, conv2_att_dst, conv2_bias),
            (conv3_W, conv3_att_src, conv3_att_dst, conv3_bias)):
        hf, a_s, a_d, m = _projection(h, w, asw, adw)
        h = _edge_phase(hf, a_s, a_d, m, srcp, dstp, zeros_np, b, h)
    return _decoder(h, out_W1, out_b1, out_W2, out_b2, out_W3, out_b3)


# speedup vs baseline: 22.1773x; 20.4002x over previous
import functools

import jax
import jax.numpy as jnp
from jax import lax
from jax.experimental import pallas as pl
from jax.experimental.pallas import tpu as pltpu
from jax.experimental.pallas import tpu_sc as plsc

N = 50000
E = 800000
ETOT = E + N
H = 4
D = 64
HD = H * D

NP = 53248
TA = 26624
EP = 32 * TA
_NBATCH = TA // 1024

_BN = 2000


def _elu(v):
    return jnp.where(v > 0, v, jnp.exp(v) - 1.0)


def _enc_body(x_ref, w1_ref, b1_ref, w2_ref, b2_ref, o_ref):
    h = jnp.dot(x_ref[...], w1_ref[...], preferred_element_type=jnp.float32)
    h = _elu(h + b1_ref[...][None, :])
    h = jnp.dot(h, w2_ref[...], preferred_element_type=jnp.float32)
    o_ref[...] = _elu(h + b2_ref[...][None, :])


def _encoder(x, w1, b1, w2, b2):
    return pl.pallas_call(
        _enc_body,
        grid=(N // _BN,),
        in_specs=[
            pl.BlockSpec((_BN, 8), lambda i: (i, 0)),
            pl.BlockSpec((8, 32), lambda i: (0, 0)),
            pl.BlockSpec((32,), lambda i: (0,)),
            pl.BlockSpec((32, D), lambda i: (0, 0)),
            pl.BlockSpec((D,), lambda i: (0,)),
        ],
        out_specs=pl.BlockSpec((_BN, D), lambda i: (i, 0)),
        out_shape=jax.ShapeDtypeStruct((N, D), jnp.float32),
    )(x, w1, b1, w2, b2)


def _proj_body(h_ref, w_ref, asw_ref, adw_ref, hf_ref, as_ref, ad_ref, m_ref):
    i = pl.program_id(0)
    hf = jnp.dot(h_ref[...], w_ref[...], preferred_element_type=jnp.float32)
    hf_ref[...] = hf
    hh = hf.reshape(_BN, H, D)
    a_s = jnp.sum(hh * asw_ref[...][None, :, :], axis=-1)
    a_d = jnp.sum(hh * adw_ref[...][None, :, :], axis=-1)
    as_ref[...] = a_s
    ad_ref[...] = a_d
    blk = jnp.concatenate([jnp.max(a_s, axis=0), jnp.max(a_d, axis=0),
                           jnp.zeros((24,), jnp.float32)])

    @pl.when(i == 0)
    def _():
        m_ref[...] = blk

    @pl.when(i > 0)
    def _():
        m_ref[...] = jnp.maximum(m_ref[...], blk)


def _projection(h, w, att_src, att_dst):
    return pl.pallas_call(
        _proj_body,
        grid=(N // _BN,),
        in_specs=[
            pl.BlockSpec((_BN, D), lambda i: (i, 0)),
            pl.BlockSpec((D, HD), lambda i: (0, 0)),
            pl.BlockSpec((H, D), lambda i: (0, 0)),
            pl.BlockSpec((H, D), lambda i: (0, 0)),
        ],
        out_specs=[
            pl.BlockSpec((_BN, HD), lambda i: (i, 0)),
            pl.BlockSpec((_BN, H), lambda i: (i, 0)),
            pl.BlockSpec((_BN, H), lambda i: (i, 0)),
            pl.BlockSpec((32,), lambda i: (0,)),
        ],
        out_shape=[
            jax.ShapeDtypeStruct((N, HD), jnp.float32),
            jax.ShapeDtypeStruct((N, H), jnp.float32),
            jax.ShapeDtypeStruct((N, H), jnp.float32),
            jax.ShapeDtypeStruct((32,), jnp.float32),
        ],
    )(h, w, att_src.reshape(H, D), att_dst.reshape(H, D))


def _dec_body(h_ref, w1_ref, b1_ref, w2_ref, b2_ref, w3_ref, b3_ref, o_ref):
    o = _elu(jnp.dot(h_ref[...], w1_ref[...],
                     preferred_element_type=jnp.float32) + b1_ref[...][None, :])
    o = _elu(jnp.dot(o, w2_ref[...],
                     preferred_element_type=jnp.float32) + b2_ref[...][None, :])
    o_ref[...] = jnp.dot(o, w3_ref[...],
                         preferred_element_type=jnp.float32) + b3_ref[...][None, :]


def _decoder(h, w1, b1, w2, b2, w3, b3):
    return pl.pallas_call(
        _dec_body,
        grid=(N // _BN,),
        in_specs=[
            pl.BlockSpec((_BN, D), lambda i: (i, 0)),
            pl.BlockSpec((D, 64), lambda i: (0, 0)),
            pl.BlockSpec((64,), lambda i: (0,)),
            pl.BlockSpec((64, 32), lambda i: (0, 0)),
            pl.BlockSpec((32,), lambda i: (0,)),
            pl.BlockSpec((32, 8), lambda i: (0, 0)),
            pl.BlockSpec((8,), lambda i: (0,)),
        ],
        out_specs=pl.BlockSpec((_BN, 8), lambda i: (i, 0)),
        out_shape=jax.ShapeDtypeStruct((N, 8), jnp.float32),
    )(h, w1, b1, w2, b2, w3, b3)


_SC_MESH = plsc.VectorSubcoreMesh(core_axis_name="c", subcore_axis_name="s")


@functools.partial(
    pl.kernel,
    out_type=[
        jax.ShapeDtypeStruct((H, EP), jnp.float32),
        jax.ShapeDtypeStruct((2 * H * NP,), jnp.float32),
    ],
    mesh=_SC_MESH,
    compiler_params=pltpu.CompilerParams(use_tc_tiling_on_sc=False),
    scratch_types=[
        pltpu.VMEM((1024,), jnp.int32),
        pltpu.VMEM((1024,), jnp.int32),
        pltpu.VMEM((H, 1024), jnp.float32),
        pltpu.VMEM((H, 1024), jnp.float32),
        pltpu.VMEM((H, 1024), jnp.float32),
        pltpu.VMEM((32,), jnp.float32),
        pltpu.VMEM_SHARED((H, NP), jnp.float32),
        pltpu.SemaphoreType.DMA,
        pltpu.SemaphoreType.DMA,
    ],
)
def _attn_sc(src_ref, dst_ref, asrc_ref, adst_ref, m_ref, zeros_ref,
             ex_ref, den_ref,
             srcb, dstb, arow, brow, exb, mb, den_acc, sem1, sem2):
    c = lax.axis_index("c")
    s = lax.axis_index("s")
    wid = s * 2 + c
    rows = NP // 16
    for h in range(H):
        pltpu.sync_copy(zeros_ref.at[pl.ds(s * rows, rows)],
                        den_acc.at[h].at[pl.ds(s * rows, rows)])
    pltpu.sync_copy(m_ref, mb)
    plsc.subcore_barrier()

    mv = mb[pl.ds(0, 16)]
    mh = []
    for h in range(H):
        v = mv[h] + mv[H + h]
        mh.append(jnp.where(v > 0, v, 0.2 * v))
    base0 = wid * TA

    def batch(b, carry):
        base = base0 + b * 1024
        pltpu.sync_copy(src_ref.at[pl.ds(base, 1024)], srcb)
        pltpu.sync_copy(dst_ref.at[pl.ds(base, 1024)], dstb)
        cps = []
        for h in range(H):
            cps.append(pltpu.async_copy(asrc_ref.at[h].at[srcb],
                                        arow.at[h], sem1))
            cps.append(pltpu.async_copy(adst_ref.at[h].at[dstb],
                                        brow.at[h], sem2))
        for cp in cps:
            cp.wait()

        def vec(j, carry2):
            sl = pl.ds(j * 16, 16)
            for h in range(H):
                e = arow[h, sl] + brow[h, sl]
                e = jnp.where(e > 0, e, 0.2 * e) - mh[h]
                exb[h, sl] = jnp.exp(e)
            return carry2

        lax.fori_loop(0, 64, vec, 0)
        for h in range(H):
            pltpu.sync_copy(exb.at[h], ex_ref.at[h].at[pl.ds(base, 1024)])
            pltpu.sync_copy(exb.at[h], den_acc.at[h].at[dstb], add=True)
        return carry

    lax.fori_loop(0, _NBATCH, batch, 0)
    plsc.subcore_barrier()
    for h in range(H):
        pltpu.sync_copy(den_acc.at[h].at[pl.ds(s * rows, rows)],
                        den_ref.at[pl.ds((c * H + h) * NP + s * rows, rows)])


_CH = 4096
_NCHUNK = NP // _CH
_TS = EP // 16
_NB_B = _TS // 1024
_G = 64


@functools.partial(
    pl.kernel,
    out_type=jax.ShapeDtypeStruct((NP, D), jnp.float32),
    mesh=_SC_MESH,
    compiler_params=pltpu.CompilerParams(use_tc_tiling_on_sc=False,
                                         needs_layout_passes=False),
    scratch_types=[
        pltpu.VMEM((1024,), jnp.int32),
        pltpu.VMEM((1024,), jnp.int32),
        pltpu.VMEM((H * 1024,), jnp.float32),
        pltpu.VMEM((H * _CH,), jnp.float32),
        pltpu.VMEM((_CH,), jnp.float32),
        pltpu.VMEM((1152,), jnp.int32),
        pltpu.VMEM((1152,), jnp.int32),
        pltpu.VMEM((H * 1152,), jnp.float32),
        pltpu.VMEM((_G, HD), jnp.float32),
        pltpu.VMEM((_G,), jnp.int32),
        pltpu.VMEM((32, HD), jnp.float32),
        pltpu.VMEM((32, D), jnp.float32),
        pltpu.VMEM((32, D), jnp.float32),
        pltpu.VMEM((D,), jnp.float32),
        pltpu.VMEM_SHARED((_CH, HD), jnp.float32),
        pltpu.SemaphoreType.DMA,
    ],
)
def _agg_sc(hf_ref, ex_ref, src_ref, dst_ref, den_ref, hprev_ref, bias_ref,
            hout_ref,
            srcb, dstb, exb, rden, tmp, src_st, dstl_st, alpha_st,
            hrows, idxg, aggb, hpb, outb, bb, agg, sem):
    cid = lax.axis_index("c")
    s = lax.axis_index("s")
    pltpu.sync_copy(bias_ref, bb)
    lane = lax.iota(jnp.int32, 16)
    zero16 = jnp.zeros((16,), jnp.float32)

    def chunk_body(ci, carry0):
        chunk = cid + 2 * ci

        @pl.when(chunk < _NCHUNK)
        def _():
            base_node = chunk * _CH

            def zr(r, cz):
                for v in range(16):
                    hrows[r, pl.ds(v * 16, 16)] = zero16
                return cz

            lax.fori_loop(0, _G, zr, 0)
            for zc in range(4):
                pltpu.sync_copy(hrows, agg.at[pl.ds(s * 256 + zc * _G, _G)])

            izero16 = jnp.zeros((16,), jnp.int32)

            def zst(t, cz):
                src_st[pl.ds(t * 16, 16)] = izero16
                dstl_st[pl.ds(t * 16, 16)] = izero16
                return cz

            lax.fori_loop(0, 72, zst, 0)
            for h in range(H):
                pltpu.sync_copy(den_ref.at[pl.ds(h * NP + base_node, _CH)],
                                rden.at[pl.ds(h * _CH, _CH)])
                pltpu.sync_copy(
                    den_ref.at[pl.ds((H + h) * NP + base_node, _CH)], tmp)

                def rd(r, c2):
                    slr = pl.ds(h * _CH + r * 16, 16)
                    rden[slr] = 1.0 / (rden[slr] + tmp[pl.ds(r * 16, 16)]
                                       + 1e-16)
                    return c2

                lax.fori_loop(0, _CH // 16, rd, 0)
            plsc.subcore_barrier()

            def flush(k, cnt_base):
                pltpu.async_copy(
                    hf_ref.at[src_st.at[pl.ds(k * _G, _G)]], hrows, sem
                ).wait()
                for t in range(4):
                    idxg[pl.ds(16 * t, 16)] = dstl_st[pl.ds(k * _G + 16 * t,
                                                            16)]

                def medge(e, c3):
                    for h in range(H):
                        idxv = jnp.full((16,), h * 1152 + k * _G + e,
                                        jnp.int32)
                        av = plsc.load_gather(alpha_st, [idxv])
                        for v in range(4):
                            sl = pl.ds(h * D + v * 16, 16)
                            hrows[e, sl] = hrows[e, sl] * av
                    return c3

                lax.fori_loop(0, _G, medge, 0)
                pltpu.sync_copy(hrows, agg.at[idxg], add=True)
                return cnt_base

            def batch(b, cnt):
                base = s * _TS + b * 1024
                pltpu.sync_copy(src_ref.at[pl.ds(base, 1024)], srcb)
                pltpu.sync_copy(dst_ref.at[pl.ds(base, 1024)], dstb)
                for h in range(H):
                    pltpu.sync_copy(ex_ref.at[h].at[pl.ds(base, 1024)],
                                    exb.at[pl.ds(h * 1024, 1024)])

                def vec(j, cnt2):
                    sl = pl.ds(j * 16, 16)
                    dstl = dstb[sl] - base_node
                    mask = (dstl >= 0) & (dstl < _CH)
                    dstl_c = jnp.minimum(jnp.maximum(dstl, 0), _CH - 1)
                    plsc.store_compressed(src_st.at[pl.ds(cnt2, 16)],
                                          srcb[sl], mask=mask)
                    plsc.store_compressed(dstl_st.at[pl.ds(cnt2, 16)],
                                          dstl_c, mask=mask)
                    for h in range(H):
                        a_h = (exb[pl.ds(h * 1024 + j * 16, 16)]
                               * plsc.load_gather(rden, [dstl_c + h * _CH]))
                        plsc.store_compressed(
                            alpha_st.at[pl.ds(h * 1152 + cnt2, 16)], a_h,
                            mask=mask)
                    pc = plsc.all_reduce_population_count(mask)
                    return cnt2 + pc[0]

                cnt = lax.fori_loop(0, 64, vec, cnt)
                kfull = cnt >> 6
                lax.fori_loop(0, kfull, flush, 0)

                @pl.when(kfull > 0)
                def _():
                    for t in range(4):
                        so = pl.ds(kfull * _G + 16 * t, 16)
                        do = pl.ds(16 * t, 16)
                        src_st[do] = src_st[so]
                        dstl_st[do] = dstl_st[so]
                        for h in range(H):
                            alpha_st[pl.ds(h * 1152 + 16 * t, 16)] = (
                                alpha_st[pl.ds(h * 1152 + kfull * _G + 16 * t,
                                               16)])

                return cnt - kfull * _G

            cnt = lax.fori_loop(0, _NB_B, batch, 0)

            @pl.when(cnt > 0)
            def _():
                for h in range(H):
                    for t in range(4):
                        alpha_st[pl.ds(h * 1152 + cnt + 16 * t, 16)] = zero16
                flush(0, 0)

            plsc.subcore_barrier()

            for w in range(8):
                loc = s * 256 + w * 32
                g0 = base_node + loc
                pltpu.sync_copy(agg.at[pl.ds(loc, 32)], aggb)
                pltpu.sync_copy(hprev_ref.at[pl.ds(g0, 32)], hpb)

                def wout(n, c4):
                    for v in range(4):
                        sl = pl.ds(v * 16, 16)
                        acc = (aggb[n, pl.ds(v * 16, 16)]
                               + aggb[n, pl.ds(D + v * 16, 16)]
                               + aggb[n, pl.ds(2 * D + v * 16, 16)]
                               + aggb[n, pl.ds(3 * D + v * 16, 16)])
                        outb[n, sl] = acc * 0.25 + bb[sl] + hpb[n, sl]
                    return c4

                lax.fori_loop(0, 32, wout, 0)
                pltpu.sync_copy(outb, hout_ref.at[pl.ds(g0, 32)])

        return carry0

    lax.fori_loop(0, 7, chunk_body, 0)


def _edge_phase(hf, a_src, a_dst, m_bound, srcp, dstp, zeros_np,
                bias, h_prev_p):
    asrc_t = a_src.T
    adst_t = jnp.pad(a_dst.T, ((0, 0), (0, NP - N)))
    ex, den = _attn_sc(srcp, dstp, asrc_t, adst_t, m_bound, zeros_np)
    return _agg_sc(hf, ex, srcp, dstp, den, h_prev_p, bias)


def kernel(x, edge_index, enc_W1, enc_b1, enc_W2, enc_b2,
           conv1_W, conv1_att_src, conv1_att_dst, conv1_bias,
           conv2_W, conv2_att_src, conv2_att_dst, conv2_bias,
           conv3_W, conv3_att_src, conv3_att_dst, conv3_bias,
           out_W1, out_b1, out_W2, out_b2, out_W3, out_b3):
    loop = jnp.arange(N, dtype=edge_index.dtype)
    srcp = jnp.concatenate([edge_index[0], loop,
                            jnp.zeros((EP - ETOT,), edge_index.dtype)])
    dstp = jnp.concatenate([edge_index[1], loop,
                            jnp.full((EP - ETOT,), N, edge_index.dtype)])
    zeros_np = jnp.zeros((NP,), jnp.float32)

    h = _encoder(x, enc_W1, enc_b1, enc_W2, enc_b2)
    hp = jnp.pad(h, ((0, NP - N), (0, 0)))
    for (w, asw, adw, b) in (
            (conv1_W, conv1_att_src, conv1_att_dst, conv1_bias),
            (conv2_W, conv2_att_src, conv2_att_dst, conv2_bias),
            (conv3_W, conv3_att_src, conv3_att_dst, conv3_bias)):
        hf, a_s, a_d, m = _projection(hp, w, asw, adw)
        hp = _edge_phase(hf, a_s, a_d, m, srcp, dstp, zeros_np,
                         b, hp)
    return _decoder(hp, out_W1, out_b1, out_W2, out_b2, out_W3, out_b3)


# speedup vs baseline: 27.3097x; 1.2314x over previous
import functools

import jax
import jax.numpy as jnp
from jax import lax
from jax.experimental import pallas as pl
from jax.experimental.pallas import tpu as pltpu
from jax.experimental.pallas import tpu_sc as plsc

N = 50000
E = 800000
ETOT = E + N
H = 4
D = 64
HD = H * D

NP = 53248
TA = 26624
EP = 32 * TA
_NBATCH = TA // 1024

_BN = 2000


def _elu(v):
    return jnp.where(v > 0, v, jnp.exp(v) - 1.0)


def _enc_body(x_ref, w1_ref, b1_ref, w2_ref, b2_ref, o_ref):
    h = jnp.dot(x_ref[...], w1_ref[...], preferred_element_type=jnp.float32)
    h = _elu(h + b1_ref[...][None, :])
    h = jnp.dot(h, w2_ref[...], preferred_element_type=jnp.float32)
    o_ref[...] = _elu(h + b2_ref[...][None, :])


def _encoder(x, w1, b1, w2, b2):
    return pl.pallas_call(
        _enc_body,
        grid=(N // _BN,),
        in_specs=[
            pl.BlockSpec((_BN, 8), lambda i: (i, 0)),
            pl.BlockSpec((8, 32), lambda i: (0, 0)),
            pl.BlockSpec((32,), lambda i: (0,)),
            pl.BlockSpec((32, D), lambda i: (0, 0)),
            pl.BlockSpec((D,), lambda i: (0,)),
        ],
        out_specs=pl.BlockSpec((_BN, D), lambda i: (i, 0)),
        out_shape=jax.ShapeDtypeStruct((N, D), jnp.float32),
    )(x, w1, b1, w2, b2)


def _proj_body(h_ref, w_ref, asw_ref, adw_ref, hf_ref, as_ref, ad_ref, m_ref):
    i = pl.program_id(0)
    hf = jnp.dot(h_ref[...], w_ref[...], preferred_element_type=jnp.float32)
    hf_ref[...] = hf
    hh = hf.reshape(_BN, H, D)
    a_s = jnp.sum(hh * asw_ref[...][None, :, :], axis=-1)
    a_d = jnp.sum(hh * adw_ref[...][None, :, :], axis=-1)
    as_ref[...] = a_s
    ad_ref[...] = a_d
    blk = jnp.concatenate([jnp.max(a_s, axis=0), jnp.max(a_d, axis=0),
                           jnp.zeros((24,), jnp.float32)])

    @pl.when(i == 0)
    def _():
        m_ref[...] = blk

    @pl.when(i > 0)
    def _():
        m_ref[...] = jnp.maximum(m_ref[...], blk)


def _projection(h, w, att_src, att_dst):
    return pl.pallas_call(
        _proj_body,
        grid=(N // _BN,),
        in_specs=[
            pl.BlockSpec((_BN, D), lambda i: (i, 0)),
            pl.BlockSpec((D, HD), lambda i: (0, 0)),
            pl.BlockSpec((H, D), lambda i: (0, 0)),
            pl.BlockSpec((H, D), lambda i: (0, 0)),
        ],
        out_specs=[
            pl.BlockSpec((_BN, HD), lambda i: (i, 0)),
            pl.BlockSpec((_BN, H), lambda i: (i, 0)),
            pl.BlockSpec((_BN, H), lambda i: (i, 0)),
            pl.BlockSpec((32,), lambda i: (0,)),
        ],
        out_shape=[
            jax.ShapeDtypeStruct((N, HD), jnp.float32),
            jax.ShapeDtypeStruct((N, H), jnp.float32),
            jax.ShapeDtypeStruct((N, H), jnp.float32),
            jax.ShapeDtypeStruct((32,), jnp.float32),
        ],
    )(h, w, att_src.reshape(H, D), att_dst.reshape(H, D))


def _dec_body(h_ref, w1_ref, b1_ref, w2_ref, b2_ref, w3_ref, b3_ref, o_ref):
    o = _elu(jnp.dot(h_ref[...], w1_ref[...],
                     preferred_element_type=jnp.float32) + b1_ref[...][None, :])
    o = _elu(jnp.dot(o, w2_ref[...],
                     preferred_element_type=jnp.float32) + b2_ref[...][None, :])
    o_ref[...] = jnp.dot(o, w3_ref[...],
                         preferred_element_type=jnp.float32) + b3_ref[...][None, :]


def _decoder(h, w1, b1, w2, b2, w3, b3):
    return pl.pallas_call(
        _dec_body,
        grid=(N // _BN,),
        in_specs=[
            pl.BlockSpec((_BN, D), lambda i: (i, 0)),
            pl.BlockSpec((D, 64), lambda i: (0, 0)),
            pl.BlockSpec((64,), lambda i: (0,)),
            pl.BlockSpec((64, 32), lambda i: (0, 0)),
            pl.BlockSpec((32,), lambda i: (0,)),
            pl.BlockSpec((32, 8), lambda i: (0, 0)),
            pl.BlockSpec((8,), lambda i: (0,)),
        ],
        out_specs=pl.BlockSpec((_BN, 8), lambda i: (i, 0)),
        out_shape=jax.ShapeDtypeStruct((N, 8), jnp.float32),
    )(h, w1, b1, w2, b2, w3, b3)


_SC_MESH = plsc.VectorSubcoreMesh(core_axis_name="c", subcore_axis_name="s")


@functools.partial(
    pl.kernel,
    out_type=[
        jax.ShapeDtypeStruct((H, EP), jnp.float32),
        jax.ShapeDtypeStruct((2 * H * NP,), jnp.float32),
    ],
    mesh=_SC_MESH,
    compiler_params=pltpu.CompilerParams(use_tc_tiling_on_sc=False),
    scratch_types=[
        pltpu.VMEM((1024,), jnp.int32),
        pltpu.VMEM((1024,), jnp.int32),
        pltpu.VMEM((H, 1024), jnp.float32),
        pltpu.VMEM((H, 1024), jnp.float32),
        pltpu.VMEM((H, 1024), jnp.float32),
        pltpu.VMEM((32,), jnp.float32),
        pltpu.VMEM_SHARED((H, NP), jnp.float32),
        pltpu.SemaphoreType.DMA,
        pltpu.SemaphoreType.DMA,
    ],
)
def _attn_sc(src_ref, dst_ref, asrc_ref, adst_ref, m_ref, zeros_ref,
             ex_ref, den_ref,
             srcb, dstb, arow, brow, exb, mb, den_acc, sem1, sem2):
    c = lax.axis_index("c")
    s = lax.axis_index("s")
    wid = s * 2 + c
    rows = NP // 16
    for h in range(H):
        pltpu.sync_copy(zeros_ref.at[pl.ds(s * rows, rows)],
                        den_acc.at[h].at[pl.ds(s * rows, rows)])
    pltpu.sync_copy(m_ref, mb)
    plsc.subcore_barrier()

    mv = mb[pl.ds(0, 16)]
    mh = []
    for h in range(H):
        v = mv[h] + mv[H + h]
        mh.append(jnp.where(v > 0, v, 0.2 * v))
    base0 = wid * TA

    def batch(b, carry):
        base = base0 + b * 1024
        pltpu.sync_copy(src_ref.at[pl.ds(base, 1024)], srcb)
        pltpu.sync_copy(dst_ref.at[pl.ds(base, 1024)], dstb)
        cps = []
        for h in range(H):
            cps.append(pltpu.async_copy(asrc_ref.at[h].at[srcb],
                                        arow.at[h], sem1))
            cps.append(pltpu.async_copy(adst_ref.at[h].at[dstb],
                                        brow.at[h], sem2))
        for cp in cps:
            cp.wait()

        def vec(j, carry2):
            sl = pl.ds(j * 16, 16)
            for h in range(H):
                e = arow[h, sl] + brow[h, sl]
                e = jnp.where(e > 0, e, 0.2 * e) - mh[h]
                exb[h, sl] = jnp.exp(e)
            return carry2

        lax.fori_loop(0, 64, vec, 0)
        for h in range(H):
            pltpu.sync_copy(exb.at[h], ex_ref.at[h].at[pl.ds(base, 1024)])
            pltpu.sync_copy(exb.at[h], den_acc.at[h].at[dstb], add=True)
        return carry

    lax.fori_loop(0, _NBATCH, batch, 0)
    plsc.subcore_barrier()
    for h in range(H):
        pltpu.sync_copy(den_acc.at[h].at[pl.ds(s * rows, rows)],
                        den_ref.at[pl.ds((c * H + h) * NP + s * rows, rows)])


_CH = 4096
_NCHUNK = NP // _CH
_TS = EP // 16
_NB_B = _TS // 1024
_G = 64


@functools.partial(
    pl.kernel,
    out_type=jax.ShapeDtypeStruct((NP, D), jnp.float32),
    mesh=_SC_MESH,
    compiler_params=pltpu.CompilerParams(use_tc_tiling_on_sc=False,
                                         needs_layout_passes=False),
    scratch_types=[
        pltpu.VMEM((1024,), jnp.int32),
        pltpu.VMEM((1024,), jnp.int32),
        pltpu.VMEM((H * 1024,), jnp.float32),
        pltpu.VMEM((H * _CH,), jnp.float32),
        pltpu.VMEM((_CH,), jnp.float32),
        pltpu.VMEM((1152,), jnp.int32),
        pltpu.VMEM((1152,), jnp.int32),
        pltpu.VMEM((H * 1152,), jnp.float32),
        pltpu.VMEM((_G, HD), jnp.float32),
        pltpu.VMEM((_G,), jnp.int32),
        pltpu.VMEM((32, HD), jnp.float32),
        pltpu.VMEM((32, D), jnp.float32),
        pltpu.VMEM((32, D), jnp.float32),
        pltpu.VMEM((D,), jnp.float32),
        pltpu.VMEM_SHARED((_CH, HD), jnp.float32),
        pltpu.SemaphoreType.DMA,
    ],
)
def _agg_sc(hf_ref, ex_ref, src_ref, dst_ref, den_ref, hprev_ref, bias_ref,
            hout_ref,
            srcb, dstb, exb, rden, tmp, src_st, dstl_st, alpha_st,
            hrows, idxg, aggb, hpb, outb, bb, agg, sem):
    cid = lax.axis_index("c")
    s = lax.axis_index("s")
    pltpu.sync_copy(bias_ref, bb)
    lane = lax.iota(jnp.int32, 16)
    zero16 = jnp.zeros((16,), jnp.float32)

    def chunk_body(ci, carry0):
        chunk = cid + 2 * ci

        @pl.when(chunk < _NCHUNK)
        def _():
            base_node = chunk * _CH

            def zr(r, cz):
                for v in range(16):
                    hrows[r, pl.ds(v * 16, 16)] = zero16
                return cz

            lax.fori_loop(0, _G, zr, 0)
            for zc in range(4):
                pltpu.sync_copy(hrows, agg.at[pl.ds(s * 256 + zc * _G, _G)])

            izero16 = jnp.zeros((16,), jnp.int32)

            def zst(t, cz):
                src_st[pl.ds(t * 16, 16)] = izero16
                dstl_st[pl.ds(t * 16, 16)] = izero16
                return cz

            lax.fori_loop(0, 72, zst, 0)
            for h in range(H):
                pltpu.sync_copy(den_ref.at[pl.ds(h * NP + base_node, _CH)],
                                rden.at[pl.ds(h * _CH, _CH)])
                pltpu.sync_copy(
                    den_ref.at[pl.ds((H + h) * NP + base_node, _CH)], tmp)

                def rd(r, c2):
                    slr = pl.ds(h * _CH + r * 16, 16)
                    rden[slr] = 1.0 / (rden[slr] + tmp[pl.ds(r * 16, 16)]
                                       + 1e-16)
                    return c2

                lax.fori_loop(0, _CH // 16, rd, 0)
            plsc.subcore_barrier()

            def flush(k, cnt_base):
                pltpu.async_copy(
                    hf_ref.at[src_st.at[pl.ds(k * _G, _G)]], hrows, sem
                ).wait()
                for t in range(4):
                    idxg[pl.ds(16 * t, 16)] = dstl_st[pl.ds(k * _G + 16 * t,
                                                            16)]

                def medge(e, c3):
                    for h in range(H):
                        idxv = jnp.full((16,), h * 1152 + k * _G + e,
                                        jnp.int32)
                        av = plsc.load_gather(alpha_st, [idxv])
                        for v in range(4):
                            sl = pl.ds(h * D + v * 16, 16)
                            hrows[e, sl] = hrows[e, sl] * av
                    return c3

                lax.fori_loop(0, _G, medge, 0)
                pltpu.sync_copy(hrows, agg.at[idxg], add=True)
                return cnt_base

            def batch(b, cnt):
                base = s * _TS + b * 1024
                cps = [
                    pltpu.async_copy(src_ref.at[pl.ds(base, 1024)], srcb,
                                     sem),
                    pltpu.async_copy(dst_ref.at[pl.ds(base, 1024)], dstb,
                                     sem),
                ]
                for h in range(H):
                    cps.append(pltpu.async_copy(
                        ex_ref.at[h].at[pl.ds(base, 1024)],
                        exb.at[pl.ds(h * 1024, 1024)], sem))
                for cp in cps:
                    cp.wait()

                def vec(j, cnt2):
                    sl = pl.ds(j * 16, 16)
                    dstl = dstb[sl] - base_node
                    mask = (dstl >= 0) & (dstl < _CH)
                    dstl_c = jnp.minimum(jnp.maximum(dstl, 0), _CH - 1)
                    plsc.store_compressed(src_st.at[pl.ds(cnt2, 16)],
                                          srcb[sl], mask=mask)
                    plsc.store_compressed(dstl_st.at[pl.ds(cnt2, 16)],
                                          dstl_c, mask=mask)
                    for h in range(H):
                        a_h = (exb[pl.ds(h * 1024 + j * 16, 16)]
                               * plsc.load_gather(rden, [dstl_c + h * _CH]))
                        plsc.store_compressed(
                            alpha_st.at[pl.ds(h * 1152 + cnt2, 16)], a_h,
                            mask=mask)
                    pc = plsc.all_reduce_population_count(mask)
                    return cnt2 + pc[0]

                cnt = lax.fori_loop(0, 64, vec, cnt)
                kfull = cnt >> 6
                lax.fori_loop(0, kfull, flush, 0)

                @pl.when(kfull > 0)
                def _():
                    for t in range(4):
                        so = pl.ds(kfull * _G + 16 * t, 16)
                        do = pl.ds(16 * t, 16)
                        src_st[do] = src_st[so]
                        dstl_st[do] = dstl_st[so]
                        for h in range(H):
                            alpha_st[pl.ds(h * 1152 + 16 * t, 16)] = (
                                alpha_st[pl.ds(h * 1152 + kfull * _G + 16 * t,
                                               16)])

                return cnt - kfull * _G

            cnt = lax.fori_loop(0, _NB_B, batch, 0)

            @pl.when(cnt > 0)
            def _():
                for h in range(H):
                    for t in range(4):
                        alpha_st[pl.ds(h * 1152 + cnt + 16 * t, 16)] = zero16
                flush(0, 0)

            plsc.subcore_barrier()

            for w in range(8):
                loc = s * 256 + w * 32
                g0 = base_node + loc
                pltpu.sync_copy(agg.at[pl.ds(loc, 32)], aggb)
                pltpu.sync_copy(hprev_ref.at[pl.ds(g0, 32)], hpb)

                def wout(n, c4):
                    for v in range(4):
                        sl = pl.ds(v * 16, 16)
                        acc = (aggb[n, pl.ds(v * 16, 16)]
                               + aggb[n, pl.ds(D + v * 16, 16)]
                               + aggb[n, pl.ds(2 * D + v * 16, 16)]
                               + aggb[n, pl.ds(3 * D + v * 16, 16)])
                        outb[n, sl] = acc * 0.25 + bb[sl] + hpb[n, sl]
                    return c4

                lax.fori_loop(0, 32, wout, 0)
                pltpu.sync_copy(outb, hout_ref.at[pl.ds(g0, 32)])

        return carry0

    lax.fori_loop(0, 7, chunk_body, 0)


def _edge_phase(hf, a_src, a_dst, m_bound, srcp, dstp, zeros_np,
                bias, h_prev_p):
    asrc_t = a_src.T
    adst_t = jnp.pad(a_dst.T, ((0, 0), (0, NP - N)))
    ex, den = _attn_sc(srcp, dstp, asrc_t, adst_t, m_bound, zeros_np)
    return _agg_sc(hf, ex, srcp, dstp, den, h_prev_p, bias)


def kernel(x, edge_index, enc_W1, enc_b1, enc_W2, enc_b2,
           conv1_W, conv1_att_src, conv1_att_dst, conv1_bias,
           conv2_W, conv2_att_src, conv2_att_dst, conv2_bias,
           conv3_W, conv3_att_src, conv3_att_dst, conv3_bias,
           out_W1, out_b1, out_W2, out_b2, out_W3, out_b3):
    loop = jnp.arange(N, dtype=edge_index.dtype)
    srcp = jnp.concatenate([edge_index[0], loop,
                            jnp.zeros((EP - ETOT,), edge_index.dtype)])
    dstp = jnp.concatenate([edge_index[1], loop,
                            jnp.full((EP - ETOT,), N, edge_index.dtype)])
    zeros_np = jnp.zeros((NP,), jnp.float32)

    h = _encoder(x, enc_W1, enc_b1, enc_W2, enc_b2)
    hp = jnp.pad(h, ((0, NP - N), (0, 0)))
    for (w, asw, adw, b) in (
            (conv1_W, conv1_att_src, conv1_att_dst, conv1_bias),
            (conv2_W, conv2_att_src, conv2_att_dst, conv2_bias),
            (conv3_W, conv3_att_src, conv3_att_dst, conv3_bias)):
        hf, a_s, a_d, m = _projection(hp, w, asw, adw)
        hp = _edge_phase(hf, a_s, a_d, m, srcp, dstp, zeros_np,
                         b, hp)
    return _decoder(hp, out_W1, out_b1, out_W2, out_b2, out_W3, out_b3)


# speedup vs baseline: 29.2636x; 1.0715x over previous
import functools

import jax
import jax.numpy as jnp
from jax import lax
from jax.experimental import pallas as pl
from jax.experimental.pallas import tpu as pltpu
from jax.experimental.pallas import tpu_sc as plsc

N = 50000
E = 800000
ETOT = E + N
H = 4
D = 64
HD = H * D

NP = 53248
TA = 26624
EP = 32 * TA
_NBATCH = TA // 1024

_BN = 2000


def _elu(v):
    return jnp.where(v > 0, v, jnp.exp(v) - 1.0)


def _enc_body(x_ref, w1_ref, b1_ref, w2_ref, b2_ref, o_ref):
    h = jnp.dot(x_ref[...], w1_ref[...], preferred_element_type=jnp.float32)
    h = _elu(h + b1_ref[...][None, :])
    h = jnp.dot(h, w2_ref[...], preferred_element_type=jnp.float32)
    o_ref[...] = _elu(h + b2_ref[...][None, :])


def _encoder(x, w1, b1, w2, b2):
    return pl.pallas_call(
        _enc_body,
        grid=(N // _BN,),
        in_specs=[
            pl.BlockSpec((_BN, 8), lambda i: (i, 0)),
            pl.BlockSpec((8, 32), lambda i: (0, 0)),
            pl.BlockSpec((32,), lambda i: (0,)),
            pl.BlockSpec((32, D), lambda i: (0, 0)),
            pl.BlockSpec((D,), lambda i: (0,)),
        ],
        out_specs=pl.BlockSpec((_BN, D), lambda i: (i, 0)),
        out_shape=jax.ShapeDtypeStruct((N, D), jnp.float32),
    )(x, w1, b1, w2, b2)


def _proj_body(h_ref, w_ref, asw_ref, adw_ref, hf_ref, as_ref, ad_ref, m_ref):
    i = pl.program_id(0)
    hf = jnp.dot(h_ref[...], w_ref[...], preferred_element_type=jnp.float32)
    hf_ref[...] = hf
    hh = hf.reshape(_BN, H, D)
    a_s = jnp.sum(hh * asw_ref[...][None, :, :], axis=-1)
    a_d = jnp.sum(hh * adw_ref[...][None, :, :], axis=-1)
    as_ref[...] = a_s
    ad_ref[...] = a_d
    blk = jnp.concatenate([jnp.max(a_s, axis=0), jnp.max(a_d, axis=0),
                           jnp.zeros((24,), jnp.float32)])

    @pl.when(i == 0)
    def _():
        m_ref[...] = blk

    @pl.when(i > 0)
    def _():
        m_ref[...] = jnp.maximum(m_ref[...], blk)


def _projection(h, w, att_src, att_dst):
    return pl.pallas_call(
        _proj_body,
        grid=(N // _BN,),
        in_specs=[
            pl.BlockSpec((_BN, D), lambda i: (i, 0)),
            pl.BlockSpec((D, HD), lambda i: (0, 0)),
            pl.BlockSpec((H, D), lambda i: (0, 0)),
            pl.BlockSpec((H, D), lambda i: (0, 0)),
        ],
        out_specs=[
            pl.BlockSpec((_BN, HD), lambda i: (i, 0)),
            pl.BlockSpec((_BN, H), lambda i: (i, 0)),
            pl.BlockSpec((_BN, H), lambda i: (i, 0)),
            pl.BlockSpec((32,), lambda i: (0,)),
        ],
        out_shape=[
            jax.ShapeDtypeStruct((N, HD), jnp.float32),
            jax.ShapeDtypeStruct((N, H), jnp.float32),
            jax.ShapeDtypeStruct((N, H), jnp.float32),
            jax.ShapeDtypeStruct((32,), jnp.float32),
        ],
    )(h, w, att_src.reshape(H, D), att_dst.reshape(H, D))


def _dec_body(h_ref, w1_ref, b1_ref, w2_ref, b2_ref, w3_ref, b3_ref, o_ref):
    o = _elu(jnp.dot(h_ref[...], w1_ref[...],
                     preferred_element_type=jnp.float32) + b1_ref[...][None, :])
    o = _elu(jnp.dot(o, w2_ref[...],
                     preferred_element_type=jnp.float32) + b2_ref[...][None, :])
    o_ref[...] = jnp.dot(o, w3_ref[...],
                         preferred_element_type=jnp.float32) + b3_ref[...][None, :]


def _decoder(h, w1, b1, w2, b2, w3, b3):
    return pl.pallas_call(
        _dec_body,
        grid=(N // _BN,),
        in_specs=[
            pl.BlockSpec((_BN, D), lambda i: (i, 0)),
            pl.BlockSpec((D, 64), lambda i: (0, 0)),
            pl.BlockSpec((64,), lambda i: (0,)),
            pl.BlockSpec((64, 32), lambda i: (0, 0)),
            pl.BlockSpec((32,), lambda i: (0,)),
            pl.BlockSpec((32, 8), lambda i: (0, 0)),
            pl.BlockSpec((8,), lambda i: (0,)),
        ],
        out_specs=pl.BlockSpec((_BN, 8), lambda i: (i, 0)),
        out_shape=jax.ShapeDtypeStruct((N, 8), jnp.float32),
    )(h, w1, b1, w2, b2, w3, b3)


_SC_MESH = plsc.VectorSubcoreMesh(core_axis_name="c", subcore_axis_name="s")


@functools.partial(
    pl.kernel,
    out_type=[
        jax.ShapeDtypeStruct((H, EP), jnp.float32),
        jax.ShapeDtypeStruct((2 * H * NP,), jnp.float32),
    ],
    mesh=_SC_MESH,
    compiler_params=pltpu.CompilerParams(use_tc_tiling_on_sc=False),
    scratch_types=[
        pltpu.VMEM((1024,), jnp.int32),
        pltpu.VMEM((1024,), jnp.int32),
        pltpu.VMEM((H, 1024), jnp.float32),
        pltpu.VMEM((H, 1024), jnp.float32),
        pltpu.VMEM((H, 1024), jnp.float32),
        pltpu.VMEM((32,), jnp.float32),
        pltpu.VMEM_SHARED((H, NP), jnp.float32),
        pltpu.SemaphoreType.DMA,
        pltpu.SemaphoreType.DMA,
    ],
)
def _attn_sc(src_ref, dst_ref, asrc_ref, adst_ref, m_ref, zeros_ref,
             ex_ref, den_ref,
             srcb, dstb, arow, brow, exb, mb, den_acc, sem1, sem2):
    c = lax.axis_index("c")
    s = lax.axis_index("s")
    wid = s * 2 + c
    rows = NP // 16
    for h in range(H):
        pltpu.sync_copy(zeros_ref.at[pl.ds(s * rows, rows)],
                        den_acc.at[h].at[pl.ds(s * rows, rows)])
    pltpu.sync_copy(m_ref, mb)
    plsc.subcore_barrier()

    mv = mb[pl.ds(0, 16)]
    mh = []
    for h in range(H):
        v = mv[h] + mv[H + h]
        mh.append(jnp.where(v > 0, v, 0.2 * v))
    base0 = wid * TA

    def batch(b, carry):
        base = base0 + b * 1024
        pltpu.sync_copy(src_ref.at[pl.ds(base, 1024)], srcb)
        pltpu.sync_copy(dst_ref.at[pl.ds(base, 1024)], dstb)
        cps = []
        for h in range(H):
            cps.append(pltpu.async_copy(asrc_ref.at[h].at[srcb],
                                        arow.at[h], sem1))
            cps.append(pltpu.async_copy(adst_ref.at[h].at[dstb],
                                        brow.at[h], sem2))
        for cp in cps:
            cp.wait()

        def vec(j, carry2):
            sl = pl.ds(j * 16, 16)
            for h in range(H):
                e = arow[h, sl] + brow[h, sl]
                e = jnp.where(e > 0, e, 0.2 * e) - mh[h]
                exb[h, sl] = jnp.exp(e)
            return carry2

        lax.fori_loop(0, 64, vec, 0)
        for h in range(H):
            pltpu.sync_copy(exb.at[h], ex_ref.at[h].at[pl.ds(base, 1024)])
            pltpu.sync_copy(exb.at[h], den_acc.at[h].at[dstb], add=True)
        return carry

    lax.fori_loop(0, _NBATCH, batch, 0)
    plsc.subcore_barrier()
    for h in range(H):
        pltpu.sync_copy(den_acc.at[h].at[pl.ds(s * rows, rows)],
                        den_ref.at[pl.ds((c * H + h) * NP + s * rows, rows)])


_CH = 4096
_NCHUNK = NP // _CH
_TS = EP // 16
_NB_B = _TS // 1024
_G = 64


@functools.partial(
    pl.kernel,
    out_type=jax.ShapeDtypeStruct((NP, D), jnp.float32),
    mesh=_SC_MESH,
    compiler_params=pltpu.CompilerParams(use_tc_tiling_on_sc=False,
                                         needs_layout_passes=False),
    scratch_types=[
        pltpu.VMEM((1024,), jnp.int32),
        pltpu.VMEM((1024,), jnp.int32),
        pltpu.VMEM((H * 1024,), jnp.float32),
        pltpu.VMEM((1024,), jnp.int32),
        pltpu.VMEM((1024,), jnp.int32),
        pltpu.VMEM((H * 1024,), jnp.float32),
        pltpu.VMEM((H * _CH,), jnp.float32),
        pltpu.VMEM((_CH,), jnp.float32),
        pltpu.VMEM((1152,), jnp.int32),
        pltpu.VMEM((1152,), jnp.int32),
        pltpu.VMEM((H * 1152,), jnp.float32),
        pltpu.VMEM((_G, HD), jnp.float32),
        pltpu.VMEM((_G,), jnp.int32),
        pltpu.VMEM((16, HD), jnp.float32),
        pltpu.VMEM((16, D), jnp.float32),
        pltpu.VMEM((16, D), jnp.float32),
        pltpu.VMEM((D,), jnp.float32),
        pltpu.VMEM_SHARED((_CH, HD), jnp.float32),
        pltpu.SemaphoreType.DMA,
        pltpu.SemaphoreType.DMA,
        pltpu.SemaphoreType.DMA,
    ],
)
def _agg_sc(hf_ref, ex_ref, src_ref, dst_ref, den_ref, hprev_ref, bias_ref,
            hout_ref,
            srcb, dstb, exb, srcb1, dstb1, exb1, rden, tmp, src_st, dstl_st,
            alpha_st, hrows, idxg, aggb, hpb, outb, bb, agg, sem, semA, semB):
    cid = lax.axis_index("c")
    s = lax.axis_index("s")
    pltpu.sync_copy(bias_ref, bb)
    lane = lax.iota(jnp.int32, 16)
    zero16 = jnp.zeros((16,), jnp.float32)

    def chunk_body(ci, carry0):
        chunk = cid + 2 * ci

        @pl.when(chunk < _NCHUNK)
        def _():
            base_node = chunk * _CH

            def zr(r, cz):
                for v in range(16):
                    hrows[r, pl.ds(v * 16, 16)] = zero16
                return cz

            lax.fori_loop(0, _G, zr, 0)
            for zc in range(4):
                pltpu.sync_copy(hrows, agg.at[pl.ds(s * 256 + zc * _G, _G)])

            izero16 = jnp.zeros((16,), jnp.int32)

            def zst(t, cz):
                src_st[pl.ds(t * 16, 16)] = izero16
                dstl_st[pl.ds(t * 16, 16)] = izero16
                return cz

            lax.fori_loop(0, 72, zst, 0)
            for h in range(H):
                pltpu.sync_copy(den_ref.at[pl.ds(h * NP + base_node, _CH)],
                                rden.at[pl.ds(h * _CH, _CH)])
                pltpu.sync_copy(
                    den_ref.at[pl.ds((H + h) * NP + base_node, _CH)], tmp)

                def rd(r, c2):
                    slr = pl.ds(h * _CH + r * 16, 16)
                    rden[slr] = 1.0 / (rden[slr] + tmp[pl.ds(r * 16, 16)]
                                       + 1e-16)
                    return c2

                lax.fori_loop(0, _CH // 16, rd, 0)
            plsc.subcore_barrier()

            def flush(k, cnt_base):
                pltpu.async_copy(
                    hf_ref.at[src_st.at[pl.ds(k * _G, _G)]], hrows, sem
                ).wait()
                for t in range(4):
                    idxg[pl.ds(16 * t, 16)] = dstl_st[pl.ds(k * _G + 16 * t,
                                                            16)]

                def medge(e, c3):
                    for h in range(H):
                        idxv = jnp.full((16,), h * 1152 + k * _G + e,
                                        jnp.int32)
                        av = plsc.load_gather(alpha_st, [idxv])
                        for v in range(4):
                            sl = pl.ds(h * D + v * 16, 16)
                            hrows[e, sl] = hrows[e, sl] * av
                    return c3

                lax.fori_loop(0, _G, medge, 0)
                pltpu.sync_copy(hrows, agg.at[idxg], add=True)
                return cnt_base

            def fire(b, sb, db, eb, sm):
                base = s * _TS + b * 1024
                pltpu.async_copy(src_ref.at[pl.ds(base, 1024)], sb, sm)
                pltpu.async_copy(dst_ref.at[pl.ds(base, 1024)], db, sm)
                for h in range(H):
                    pltpu.async_copy(ex_ref.at[h].at[pl.ds(base, 1024)],
                                     eb.at[pl.ds(h * 1024, 1024)], sm)

            def drain(b, sb, db, eb, sm):
                base = s * _TS + b * 1024
                pltpu.make_async_copy(src_ref.at[pl.ds(base, 1024)], sb,
                                      sm).wait()
                pltpu.make_async_copy(dst_ref.at[pl.ds(base, 1024)], db,
                                      sm).wait()
                for h in range(H):
                    pltpu.make_async_copy(
                        ex_ref.at[h].at[pl.ds(base, 1024)],
                        eb.at[pl.ds(h * 1024, 1024)], sm).wait()

            def process(sb, db, eb, cnt):
                def vec(j, cnt2):
                    sl = pl.ds(j * 16, 16)
                    dstl = db[sl] - base_node
                    mask = (dstl >= 0) & (dstl < _CH)
                    dstl_c = jnp.minimum(jnp.maximum(dstl, 0), _CH - 1)
                    plsc.store_compressed(src_st.at[pl.ds(cnt2, 16)],
                                          sb[sl], mask=mask)
                    plsc.store_compressed(dstl_st.at[pl.ds(cnt2, 16)],
                                          dstl_c, mask=mask)
                    for h in range(H):
                        a_h = (eb[pl.ds(h * 1024 + j * 16, 16)]
                               * plsc.load_gather(rden, [dstl_c + h * _CH]))
                        plsc.store_compressed(
                            alpha_st.at[pl.ds(h * 1152 + cnt2, 16)], a_h,
                            mask=mask)
                    pc = plsc.all_reduce_population_count(mask)
                    return cnt2 + pc[0]

                cnt = lax.fori_loop(0, 64, vec, cnt)
                kfull = cnt >> 6
                lax.fori_loop(0, kfull, flush, 0)

                @pl.when(kfull > 0)
                def _():
                    for t in range(4):
                        so = pl.ds(kfull * _G + 16 * t, 16)
                        do = pl.ds(16 * t, 16)
                        src_st[do] = src_st[so]
                        dstl_st[do] = dstl_st[so]
                        for h in range(H):
                            alpha_st[pl.ds(h * 1152 + 16 * t, 16)] = (
                                alpha_st[pl.ds(h * 1152 + kfull * _G + 16 * t,
                                               16)])

                return cnt - kfull * _G

            def pair(i2, cnt):
                b0 = i2 * 2
                fire(b0 + 1, srcb1, dstb1, exb1, semB)
                drain(b0, srcb, dstb, exb, semA)
                cnt = process(srcb, dstb, exb, cnt)

                @pl.when(i2 + 1 < _NB_B // 2)
                def _():
                    fire(b0 + 2, srcb, dstb, exb, semA)

                drain(b0 + 1, srcb1, dstb1, exb1, semB)
                return process(srcb1, dstb1, exb1, cnt)

            fire(0, srcb, dstb, exb, semA)
            cnt = lax.fori_loop(0, _NB_B // 2, pair, 0)

            @pl.when(cnt > 0)
            def _():
                for h in range(H):
                    for t in range(4):
                        alpha_st[pl.ds(h * 1152 + cnt + 16 * t, 16)] = zero16
                flush(0, 0)

            plsc.subcore_barrier()

            for w in range(16):
                loc = s * 256 + w * 16
                g0 = base_node + loc
                pltpu.sync_copy(agg.at[pl.ds(loc, 16)], aggb)
                pltpu.sync_copy(hprev_ref.at[pl.ds(g0, 16)], hpb)

                def wout(n, c4):
                    for v in range(4):
                        sl = pl.ds(v * 16, 16)
                        acc = (aggb[n, pl.ds(v * 16, 16)]
                               + aggb[n, pl.ds(D + v * 16, 16)]
                               + aggb[n, pl.ds(2 * D + v * 16, 16)]
                               + aggb[n, pl.ds(3 * D + v * 16, 16)])
                        outb[n, sl] = acc * 0.25 + bb[sl] + hpb[n, sl]
                    return c4

                lax.fori_loop(0, 16, wout, 0)
                pltpu.sync_copy(outb, hout_ref.at[pl.ds(g0, 16)])

        return carry0

    lax.fori_loop(0, 7, chunk_body, 0)


def _edge_phase(hf, a_src, a_dst, m_bound, srcp, dstp, zeros_np,
                bias, h_prev_p):
    asrc_t = a_src.T
    adst_t = jnp.pad(a_dst.T, ((0, 0), (0, NP - N)))
    ex, den = _attn_sc(srcp, dstp, asrc_t, adst_t, m_bound, zeros_np)
    return _agg_sc(hf, ex, srcp, dstp, den, h_prev_p, bias)


def kernel(x, edge_index, enc_W1, enc_b1, enc_W2, enc_b2,
           conv1_W, conv1_att_src, conv1_att_dst, conv1_bias,
           conv2_W, conv2_att_src, conv2_att_dst, conv2_bias,
           conv3_W, conv3_att_src, conv3_att_dst, conv3_bias,
           out_W1, out_b1, out_W2, out_b2, out_W3, out_b3):
    loop = jnp.arange(N, dtype=edge_index.dtype)
    srcp = jnp.concatenate([edge_index[0], loop,
                            jnp.zeros((EP - ETOT,), edge_index.dtype)])
    dstp = jnp.concatenate([edge_index[1], loop,
                            jnp.full((EP - ETOT,), N, edge_index.dtype)])
    zeros_np = jnp.zeros((NP,), jnp.float32)

    h = _encoder(x, enc_W1, enc_b1, enc_W2, enc_b2)
    hp = jnp.pad(h, ((0, NP - N), (0, 0)))
    for (w, asw, adw, b) in (
            (conv1_W, conv1_att_src, conv1_att_dst, conv1_bias),
            (conv2_W, conv2_att_src, conv2_att_dst, conv2_bias),
            (conv3_W, conv3_att_src, conv3_att_dst, conv3_bias)):
        hf, a_s, a_d, m = _projection(hp, w, asw, adw)
        hp = _edge_phase(hf, a_s, a_d, m, srcp, dstp, zeros_np,
                         b, hp)
    return _decoder(hp, out_W1, out_b1, out_W2, out_b2, out_W3, out_b3)
